# baseline (speedup 1.0000x reference)
import functools

import jax
import jax.numpy as jnp
from jax import lax
from jax.experimental import pallas as pl
from jax.experimental.pallas import tpu as pltpu
from jax.experimental.pallas import tpu_sc as plsc

B = 8192
D = 1024
E = 8
K = 2
H = 4096
C = 1000
CP = 1024

N_PAIR = B * K
BLK_R = 512
S = N_PAIR + E * BLK_R
NB = S // BLK_R
BH = 512
NH = H // BH

NC = 2
NS = 16
NW = NC * NS
SW = S // NW
TW = B // NW

_SQRT_HALF = 0.7071067811865476


GB = 512


def _gate_body(x_ref, gw_ref, gb_ref, idx_ref, cw_ref, aux_ref, acc_ref):
    i = pl.program_id(0)
    nsteps = pl.num_programs(0)
    xb = x_ref[...].astype(jnp.bfloat16)
    gwb = gw_ref[...].astype(jnp.bfloat16)
    logits = lax.dot_general(
        xb, gwb, (((1,), (0,)), ((), ())),
        preferred_element_type=jnp.float32,
    ) + gb_ref[...]
    m = jnp.max(logits, axis=1, keepdims=True)
    ez = jnp.exp(logits - m)
    p = ez / jnp.sum(ez, axis=1, keepdims=True)
    lane = lax.broadcasted_iota(jnp.int32, p.shape, 1)
    m1 = jnp.max(p, axis=1, keepdims=True)
    i1 = jnp.min(jnp.where(p >= m1, lane, E), axis=1, keepdims=True)
    p2 = jnp.where(lane == i1, -jnp.inf, p)
    m2 = jnp.max(p2, axis=1, keepdims=True)
    i2 = jnp.min(jnp.where(p2 >= m2, lane, E), axis=1, keepdims=True)
    tot = m1 + m2
    idx_ref[...] = jnp.concatenate([i1, i2], axis=1)
    cw_ref[...] = jnp.concatenate([m1 / tot, m2 / tot], axis=1)

    routed = ((lane == i1) | (lane == i2)).astype(jnp.float32)

    @pl.when(i == 0)
    def _():
        acc_ref[...] = jnp.zeros_like(acc_ref)

    acc_ref[0, :] += jnp.sum(p, axis=0)
    acc_ref[1, :] += jnp.sum(routed, axis=0)

    @pl.when(i == nsteps - 1)
    def _():
        me = acc_ref[0, :] * (1.0 / B)
        ce = acc_ref[1, :] * (1.0 / B)
        aux_ref[...] = (jnp.float32(E) * jnp.sum(me * ce)).reshape(1, 1)


def _gate_call(x, gate_W, gate_b):
    return pl.pallas_call(
        _gate_body,
        grid=(B // GB,),
        in_specs=[
            pl.BlockSpec((GB, D), lambda i: (i, 0)),
            pl.BlockSpec((D, E), lambda i: (0, 0)),
            pl.BlockSpec((1, E), lambda i: (0, 0)),
        ],
        out_specs=[
            pl.BlockSpec((GB, K), lambda i: (i, 0)),
            pl.BlockSpec((GB, K), lambda i: (i, 0)),
            pl.BlockSpec((1, 1), lambda i: (0, 0)),
        ],
        out_shape=[
            jax.ShapeDtypeStruct((B, K), jnp.int32),
            jax.ShapeDtypeStruct((B, K), jnp.float32),
            jax.ShapeDtypeStruct((1, 1), jnp.float32),
        ],
        scratch_shapes=[pltpu.VMEM((2, E), jnp.float32)],
    )(x, gate_W, gate_b.reshape(1, E))


RQ = 128


def _route_body(pairs_ref, dest_ref, bexp_ref):
    pairs = pairs_ref[...]
    r_i = lax.broadcasted_iota(jnp.int32, (RQ, RQ), 0)
    q_i = lax.broadcasted_iota(jnp.int32, (RQ, RQ), 1)
    ut = (r_i <= q_i).astype(jnp.float32)
    sl = (q_i < r_i).astype(jnp.float32)

    masks = []
    counts = []
    for e in range(E):
        mf = (pairs == e).astype(jnp.float32)
        pre = lax.dot_general(mf, ut, (((1,), (0,)), ((), ())),
                              preferred_element_type=jnp.float32)
        s_col = pre[:, RQ - 1:RQ]
        off = lax.dot_general(sl, s_col, (((1,), (0,)), ((), ())),
                              preferred_element_type=jnp.float32)
        cum = pre + off
        masks.append((mf, cum))
        counts.append(jnp.sum(mf).astype(jnp.int32))

    starts = []
    start = jnp.int32(0)
    for e in range(E):
        starts.append(start)
        padded = ((counts[e] + (BLK_R - 1)) // BLK_R) * BLK_R
        start = start + padded

    dest = jnp.zeros((RQ, RQ), jnp.float32)
    for e in range(E):
        mf, cum = masks[e]
        dest = dest + mf * (cum - 1.0 + starts[e].astype(jnp.float32))
    dest_ref[...] = dest.astype(jnp.int32)

    blk0 = lax.broadcasted_iota(jnp.int32, (8, 128), 1) * BLK_R
    bexp = jnp.zeros((8, 128), jnp.int32)
    for e in range(E):
        lo = starts[e]
        hi = starts[e] + ((counts[e] + (BLK_R - 1)) // BLK_R) * BLK_R
        bexp = bexp + e * ((blk0 >= lo) & (blk0 < hi)).astype(jnp.int32)
    bexp_ref[...] = bexp


def _route_call(pairs2d):
    return pl.pallas_call(
        _route_body,
        out_shape=[
            jax.ShapeDtypeStruct((RQ, RQ), jnp.int32),
            jax.ShapeDtypeStruct((8, 128), jnp.int32),
        ],
    )(pairs2d)


CH = 32
DG = 512


def _route_gather_sc_body(dest_hbm, cw_hbm, x_hbm, xs_hbm, w_hbm,
                          dest_v, cw_v, src_loc, w_loc, rb0, rb1,
                          sr0, sr1, sw0, sw1):
    wid = lax.axis_index("s") * NC + lax.axis_index("c")
    lo = wid * SW
    pltpu.sync_copy(dest_hbm, dest_v)
    pltpu.sync_copy(cw_hbm, cw_v)

    zf = jnp.zeros((16,), jnp.float32)
    lane16 = lax.iota(jnp.int32, 16)

    @plsc.parallel_loop(0, SW // 16, unroll=4)
    def _zero(t):
        src_loc[pl.ds(t * 16, 16)] = (lo + t * 16 + lane16) & (B - 1)
        w_loc[pl.ds(t * 16, 16)] = zf

    @plsc.parallel_loop(0, N_PAIR // 16, unroll=4)
    def _scan(t):
        d = dest_v[pl.ds(t * 16, 16)]
        c = cw_v[pl.ds(t * 16, 16)]
        li = d - lo
        msk = (li >= 0) & (li < SW)
        tok = lax.shift_right_logical(t * 16 + lane16, 1)
        plsc.store_scatter(src_loc, [li], tok, mask=msk)
        plsc.store_scatter(w_loc, [li], c, mask=msk)

    pltpu.sync_copy(w_loc, w_hbm.at[pl.ds(lo, SW)])

    NCH = SW // CH
    rbufs = (rb0, rb1)
    rsems = (sr0, sr1)
    wsems = (sw0, sw1)
    rd = [None, None]
    wr = [None, None]
    for ch in range(NCH):
        b = ch % 2
        if wr[b] is not None:
            wr[b].wait()
        idx = src_loc.at[pl.ds(ch * CH, CH)]
        rd[b] = pltpu.async_copy(x_hbm.at[idx], rbufs[b], rsems[b])
        if ch >= 1:
            pb = 1 - b
            rd[pb].wait()
            wr[pb] = pltpu.async_copy(
                rbufs[pb], xs_hbm.at[pl.ds(lo + (ch - 1) * CH, CH)],
                wsems[pb])
    lb = (NCH - 1) % 2
    rd[lb].wait()
    wr[lb] = pltpu.async_copy(
        rbufs[lb], xs_hbm.at[pl.ds(lo + (NCH - 1) * CH, CH)], wsems[lb])
    wr[0].wait()
    wr[1].wait()


def _route_gather_sc(dest_flat, cw_flat, x):
    mesh = plsc.VectorSubcoreMesh(core_axis_name="c", subcore_axis_name="s")
    fn = pl.kernel(
        _route_gather_sc_body,
        out_type=[
            jax.ShapeDtypeStruct((S, DG), jnp.int32),
            jax.ShapeDtypeStruct((S,), jnp.float32),
        ],
        mesh=mesh,
        scratch_types=[
            pltpu.VMEM((N_PAIR,), jnp.int32),
            pltpu.VMEM((N_PAIR,), jnp.float32),
            pltpu.VMEM((SW,), jnp.int32),
            pltpu.VMEM((SW,), jnp.float32),
            pltpu.VMEM((CH, DG), jnp.int32),
            pltpu.VMEM((CH, DG), jnp.int32),
            pltpu.SemaphoreType.DMA,
            pltpu.SemaphoreType.DMA,
            pltpu.SemaphoreType.DMA,
            pltpu.SemaphoreType.DMA,
        ],
        compiler_params=pltpu.CompilerParams(needs_layout_passes=False),
    )
    return fn(dest_flat, cw_flat, x)


def _mlp_body(bexp_ref, xs_ref, w1_ref, b1_ref, w2_ref, b2_ref, wv_ref,
              out_ref, acc_ref):
    j = pl.program_id(1)
    xb = xs_ref[...]
    w1 = w1_ref[0].astype(jnp.bfloat16)
    h = lax.dot_general(xb, w1, (((1,), (0,)), ((), ())),
                        preferred_element_type=jnp.float32)
    h = h + b1_ref[0]
    h = 0.5 * h * (1.0 + lax.erf(h * _SQRT_HALF))
    hb = h.astype(jnp.bfloat16)
    w2 = w2_ref[0].astype(jnp.bfloat16)
    part = lax.dot_general(hb, w2, (((1,), (0,)), ((), ())),
                           preferred_element_type=jnp.float32)
    part = jnp.concatenate(
        [part, jnp.zeros((BLK_R, CP - C), jnp.float32)], axis=1)

    @pl.when(j == 0)
    def _():
        b2 = jnp.concatenate(
            [b2_ref[0], jnp.zeros((1, CP - C), jnp.float32)], axis=1)
        acc_ref[...] = jnp.broadcast_to(b2, (BLK_R, CP))

    acc_ref[...] += part

    @pl.when(j == NH - 1)
    def _():
        out_ref[...] = acc_ref[...] * wv_ref[...]


def _mlp_call(bexp_vec, xs, W1, b1, W2, b2, wcol):
    grid_spec = pltpu.PrefetchScalarGridSpec(
        num_scalar_prefetch=1,
        grid=(NB, NH),
        in_specs=[
            pl.BlockSpec((BLK_R, D), lambda i, j, be: (i, 0)),
            pl.BlockSpec((1, D, BH), lambda i, j, be: (be[i], 0, j)),
            pl.BlockSpec((1, 1, BH), lambda i, j, be: (be[i], 0, j)),
            pl.BlockSpec((1, BH, C), lambda i, j, be: (be[i], j, 0)),
            pl.BlockSpec((1, 1, C), lambda i, j, be: (be[i], 0, 0)),
            pl.BlockSpec((BLK_R, 1), lambda i, j, be: (i, 0)),
        ],
        out_specs=pl.BlockSpec((BLK_R, CP), lambda i, j, be: (i, 0)),
        scratch_shapes=[pltpu.VMEM((BLK_R, CP), jnp.float32)],
    )
    return pl.pallas_call(
        _mlp_body,
        grid_spec=grid_spec,
        out_shape=jax.ShapeDtypeStruct((S, CP), jnp.float32),
    )(bexp_vec, xs, W1, b1.reshape(E, 1, H), W2, b2.reshape(E, 1, C), wcol)


TCH = 16


def _combine_sc_body(outb_hbm, d0_hbm, d1_hbm, y_hbm,
                     d0_v, d1_v, r0a, r1a, r0b, r1b,
                     s0a, s1a, s0b, s1b, swa, swb):
    wid = lax.axis_index("s") * NC + lax.axis_index("c")
    t0 = wid * TW
    pltpu.sync_copy(d0_hbm.at[pl.ds(t0, TW)], d0_v)
    pltpu.sync_copy(d1_hbm.at[pl.ds(t0, TW)], d1_v)

    NCH2 = TW // TCH
    r0s = (r0a, r0b)
    r1s = (r1a, r1b)
    g0sem = (s0a, s0b)
    g1sem = (s1a, s1b)
    wsem = (swa, swb)
    g0 = [None, None]
    g1 = [None, None]
    wr = [None, None]

    def start(ch):
        b = ch % 2
        i0 = d0_v.at[pl.ds(ch * TCH, TCH)]
        i1 = d1_v.at[pl.ds(ch * TCH, TCH)]
        g0[b] = pltpu.async_copy(outb_hbm.at[i0], r0s[b], g0sem[b])
        g1[b] = pltpu.async_copy(outb_hbm.at[i1], r1s[b], g1sem[b])

    def finish(ch):
        b = ch % 2
        g0[b].wait()
        g1[b].wait()
        r0 = r0s[b]
        r1 = r1s[b]

        @plsc.parallel_loop(0, TCH * (CP // 16), unroll=4)
        def _add(t):
            row = lax.shift_right_logical(t, 6)
            col = lax.mul(lax.rem(t, CP // 16), 16)
            r0[row, pl.ds(col, 16)] += r1[row, pl.ds(col, 16)]

        wr[b] = pltpu.async_copy(r0, y_hbm.at[pl.ds(t0 + ch * TCH, TCH)],
                                 wsem[b])

    start(0)
    for ch in range(NCH2):
        b = ch % 2
        if ch + 1 < NCH2:
            if wr[1 - b] is not None:
                wr[1 - b].wait()
            start(ch + 1)
        finish(ch)
    wr[0].wait()
    wr[1].wait()


def _combine_sc(outbuf, d0, d1):
    mesh = plsc.VectorSubcoreMesh(core_axis_name="c", subcore_axis_name="s")
    fn = pl.kernel(
        _combine_sc_body,
        out_type=jax.ShapeDtypeStruct((B, CP), jnp.float32),
        mesh=mesh,
        scratch_types=[
            pltpu.VMEM((TW,), jnp.int32),
            pltpu.VMEM((TW,), jnp.int32),
            pltpu.VMEM((TCH, CP), jnp.float32),
            pltpu.VMEM((TCH, CP), jnp.float32),
            pltpu.VMEM((TCH, CP), jnp.float32),
            pltpu.VMEM((TCH, CP), jnp.float32),
            pltpu.SemaphoreType.DMA,
            pltpu.SemaphoreType.DMA,
            pltpu.SemaphoreType.DMA,
            pltpu.SemaphoreType.DMA,
            pltpu.SemaphoreType.DMA,
            pltpu.SemaphoreType.DMA,
        ],
        compiler_params=pltpu.CompilerParams(needs_layout_passes=False),
    )
    return fn(outbuf, d0, d1)


def kernel(x, gate_W, gate_b, W1, b1, W2, b2):
    idx, cw, aux = _gate_call(x, gate_W, gate_b)
    pairs2d = idx.reshape(RQ, RQ)
    dest2d, bexp2d = _route_call(pairs2d)
    dest_flat = dest2d.reshape(N_PAIR)
    cw_flat = cw.reshape(N_PAIR)
    x_i32 = lax.bitcast_convert_type(
        x.astype(jnp.bfloat16).reshape(B, DG, 2), jnp.int32)
    xs_i32, wbuf = _route_gather_sc(dest_flat, cw_flat, x_i32)
    xs = lax.bitcast_convert_type(xs_i32, jnp.bfloat16).reshape(S, D)
    bexp_vec = bexp2d[0, :NB]
    out_buf = _mlp_call(bexp_vec, xs, W1, b1, W2, b2, wbuf.reshape(S, 1))
    dmat = dest_flat.reshape(B, K)
    y_pad = _combine_sc(out_buf, dmat[:, 0], dmat[:, 1])
    return (y_pad[:, :C], aux[0, 0])


# speedup vs baseline: 1.5669x; 1.5669x over previous
import functools

import jax
import jax.numpy as jnp
from jax import lax
from jax.experimental import pallas as pl
from jax.experimental.pallas import tpu as pltpu
from jax.experimental.pallas import tpu_sc as plsc

B = 8192
D = 1024
E = 8
K = 2
H = 4096
C = 1000
CP = 1024

N_PAIR = B * K
BLK_R = 512
S = N_PAIR + E * BLK_R
NB = S // BLK_R
BH = 512
NH = H // BH

NC = 2
NS = 16
NW = NC * NS
SW = S // NW
TW = B // NW

_SQRT_HALF = 0.7071067811865476


GB = 512


def _gate_body(x_ref, gw_ref, gb_ref, idx_ref, cw_ref, aux_ref, acc_ref):
    i = pl.program_id(0)
    nsteps = pl.num_programs(0)
    xb = x_ref[...].astype(jnp.bfloat16)
    gwb = gw_ref[...].astype(jnp.bfloat16)
    logits = lax.dot_general(
        xb, gwb, (((1,), (0,)), ((), ())),
        preferred_element_type=jnp.float32,
    ) + gb_ref[...]
    m = jnp.max(logits, axis=1, keepdims=True)
    ez = jnp.exp(logits - m)
    p = ez / jnp.sum(ez, axis=1, keepdims=True)
    lane = lax.broadcasted_iota(jnp.int32, p.shape, 1)
    m1 = jnp.max(p, axis=1, keepdims=True)
    i1 = jnp.min(jnp.where(p >= m1, lane, E), axis=1, keepdims=True)
    p2 = jnp.where(lane == i1, -jnp.inf, p)
    m2 = jnp.max(p2, axis=1, keepdims=True)
    i2 = jnp.min(jnp.where(p2 >= m2, lane, E), axis=1, keepdims=True)
    tot = m1 + m2
    idx_ref[...] = jnp.concatenate([i1, i2], axis=1)
    cw_ref[...] = jnp.concatenate([m1 / tot, m2 / tot], axis=1)

    routed = ((lane == i1) | (lane == i2)).astype(jnp.float32)

    @pl.when(i == 0)
    def _():
        acc_ref[...] = jnp.zeros_like(acc_ref)

    acc_ref[0, :] += jnp.sum(p, axis=0)
    acc_ref[1, :] += jnp.sum(routed, axis=0)

    @pl.when(i == nsteps - 1)
    def _():
        me = acc_ref[0, :] * (1.0 / B)
        ce = acc_ref[1, :] * (1.0 / B)
        aux_ref[...] = (jnp.float32(E) * jnp.sum(me * ce)).reshape(1, 1)


def _gate_call(x, gate_W, gate_b):
    return pl.pallas_call(
        _gate_body,
        grid=(B // GB,),
        in_specs=[
            pl.BlockSpec((GB, D), lambda i: (i, 0)),
            pl.BlockSpec((D, E), lambda i: (0, 0)),
            pl.BlockSpec((1, E), lambda i: (0, 0)),
        ],
        out_specs=[
            pl.BlockSpec((GB, K), lambda i: (i, 0)),
            pl.BlockSpec((GB, K), lambda i: (i, 0)),
            pl.BlockSpec((1, 1), lambda i: (0, 0)),
        ],
        out_shape=[
            jax.ShapeDtypeStruct((B, K), jnp.int32),
            jax.ShapeDtypeStruct((B, K), jnp.float32),
            jax.ShapeDtypeStruct((1, 1), jnp.float32),
        ],
        scratch_shapes=[pltpu.VMEM((2, E), jnp.float32)],
    )(x, gate_W, gate_b.reshape(1, E))


RQ = 128


def _route_body(pairs_ref, dest_ref, bexp_ref):
    pairs = pairs_ref[...]
    r_i = lax.broadcasted_iota(jnp.int32, (RQ, RQ), 0)
    q_i = lax.broadcasted_iota(jnp.int32, (RQ, RQ), 1)
    ut = (r_i <= q_i).astype(jnp.float32)
    sl = (q_i < r_i).astype(jnp.float32)

    masks = []
    counts = []
    for e in range(E):
        mf = (pairs == e).astype(jnp.float32)
        pre = lax.dot_general(mf, ut, (((1,), (0,)), ((), ())),
                              preferred_element_type=jnp.float32)
        s_col = pre[:, RQ - 1:RQ]
        off = lax.dot_general(sl, s_col, (((1,), (0,)), ((), ())),
                              preferred_element_type=jnp.float32)
        cum = pre + off
        masks.append((mf, cum))
        counts.append(jnp.sum(mf).astype(jnp.int32))

    starts = []
    start = jnp.int32(0)
    for e in range(E):
        starts.append(start)
        padded = ((counts[e] + (BLK_R - 1)) // BLK_R) * BLK_R
        start = start + padded

    dest = jnp.zeros((RQ, RQ), jnp.float32)
    for e in range(E):
        mf, cum = masks[e]
        dest = dest + mf * (cum - 1.0 + starts[e].astype(jnp.float32))
    dest_ref[...] = dest.astype(jnp.int32)

    blk0 = lax.broadcasted_iota(jnp.int32, (8, 128), 1) * BLK_R
    bexp = jnp.zeros((8, 128), jnp.int32)
    for e in range(E):
        lo = starts[e]
        hi = starts[e] + ((counts[e] + (BLK_R - 1)) // BLK_R) * BLK_R
        bexp = bexp + e * ((blk0 >= lo) & (blk0 < hi)).astype(jnp.int32)
    bexp_ref[...] = bexp


def _route_call(pairs2d):
    return pl.pallas_call(
        _route_body,
        out_shape=[
            jax.ShapeDtypeStruct((RQ, RQ), jnp.int32),
            jax.ShapeDtypeStruct((8, 128), jnp.int32),
        ],
    )(pairs2d)


CH = 32
DG = 512


def _route_gather_sc_body(dest_hbm, cw_hbm, x_hbm, xs_hbm, w_hbm,
                          dest_v, cw_v, src_loc, w_loc, rb0, rb1,
                          sr0, sr1, sw0, sw1):
    wid = lax.axis_index("s") * NC + lax.axis_index("c")
    lo = wid * SW
    pltpu.sync_copy(dest_hbm, dest_v)
    pltpu.sync_copy(cw_hbm, cw_v)

    zf = jnp.zeros((16,), jnp.float32)
    lane16 = lax.iota(jnp.int32, 16)

    @plsc.parallel_loop(0, SW // 16, unroll=4)
    def _zero(t):
        src_loc[pl.ds(t * 16, 16)] = (lo + t * 16 + lane16) & (B - 1)
        w_loc[pl.ds(t * 16, 16)] = zf

    @plsc.parallel_loop(0, N_PAIR // 16, unroll=4)
    def _scan(t):
        d = dest_v[pl.ds(t * 16, 16)]
        c = cw_v[pl.ds(t * 16, 16)]
        li = d - lo
        msk = (li >= 0) & (li < SW)
        tok = lax.shift_right_logical(t * 16 + lane16, 1)
        plsc.store_scatter(src_loc, [li], tok, mask=msk)
        plsc.store_scatter(w_loc, [li], c, mask=msk)

    pltpu.sync_copy(w_loc, w_hbm.at[pl.ds(lo, SW)])

    NCH = SW // CH
    rbufs = (rb0, rb1)
    rsems = (sr0, sr1)
    wsems = (sw0, sw1)
    rd = [None, None]
    wr = [None, None]
    for ch in range(NCH):
        b = ch % 2
        if wr[b] is not None:
            wr[b].wait()
        idx = src_loc.at[pl.ds(ch * CH, CH)]
        rd[b] = pltpu.async_copy(x_hbm.at[idx], rbufs[b], rsems[b])
        if ch >= 1:
            pb = 1 - b
            rd[pb].wait()
            wr[pb] = pltpu.async_copy(
                rbufs[pb], xs_hbm.at[pl.ds(lo + (ch - 1) * CH, CH)],
                wsems[pb])
    lb = (NCH - 1) % 2
    rd[lb].wait()
    wr[lb] = pltpu.async_copy(
        rbufs[lb], xs_hbm.at[pl.ds(lo + (NCH - 1) * CH, CH)], wsems[lb])
    wr[0].wait()
    wr[1].wait()


def _route_gather_sc(dest_flat, cw_flat, x):
    mesh = plsc.VectorSubcoreMesh(core_axis_name="c", subcore_axis_name="s")
    fn = pl.kernel(
        _route_gather_sc_body,
        out_type=[
            jax.ShapeDtypeStruct((S, D), jnp.float32),
            jax.ShapeDtypeStruct((S,), jnp.float32),
        ],
        mesh=mesh,
        scratch_types=[
            pltpu.VMEM((N_PAIR,), jnp.int32),
            pltpu.VMEM((N_PAIR,), jnp.float32),
            pltpu.VMEM((SW,), jnp.int32),
            pltpu.VMEM((SW,), jnp.float32),
            pltpu.VMEM((CH, D), jnp.float32),
            pltpu.VMEM((CH, D), jnp.float32),
            pltpu.SemaphoreType.DMA,
            pltpu.SemaphoreType.DMA,
            pltpu.SemaphoreType.DMA,
            pltpu.SemaphoreType.DMA,
        ],
        compiler_params=pltpu.CompilerParams(needs_layout_passes=False),
    )
    return fn(dest_flat, cw_flat, x)


def _mlp_body(bexp_ref, xs_ref, w1_ref, b1_ref, w2_ref, b2_ref, wv_ref,
              out_ref, acc_ref):
    j = pl.program_id(1)
    xb = xs_ref[...].astype(jnp.bfloat16)
    w1 = w1_ref[0].astype(jnp.bfloat16)
    h = lax.dot_general(xb, w1, (((1,), (0,)), ((), ())),
                        preferred_element_type=jnp.float32)
    h = h + b1_ref[0]
    h = 0.5 * h * (1.0 + lax.erf(h * _SQRT_HALF))
    hb = h.astype(jnp.bfloat16)
    w2 = w2_ref[0].astype(jnp.bfloat16)
    part = lax.dot_general(hb, w2, (((1,), (0,)), ((), ())),
                           preferred_element_type=jnp.float32)
    part = jnp.concatenate(
        [part, jnp.zeros((BLK_R, CP - C), jnp.float32)], axis=1)

    @pl.when(j == 0)
    def _():
        b2 = jnp.concatenate(
            [b2_ref[0], jnp.zeros((1, CP - C), jnp.float32)], axis=1)
        acc_ref[...] = jnp.broadcast_to(b2, (BLK_R, CP))

    acc_ref[...] += part

    @pl.when(j == NH - 1)
    def _():
        out_ref[...] = acc_ref[...] * wv_ref[...]


def _mlp_call(bexp_vec, xs, W1, b1, W2, b2, wcol):
    grid_spec = pltpu.PrefetchScalarGridSpec(
        num_scalar_prefetch=1,
        grid=(NB, NH),
        in_specs=[
            pl.BlockSpec((BLK_R, D), lambda i, j, be: (i, 0)),
            pl.BlockSpec((1, D, BH), lambda i, j, be: (be[i], 0, j)),
            pl.BlockSpec((1, 1, BH), lambda i, j, be: (be[i], 0, j)),
            pl.BlockSpec((1, BH, C), lambda i, j, be: (be[i], j, 0)),
            pl.BlockSpec((1, 1, C), lambda i, j, be: (be[i], 0, 0)),
            pl.BlockSpec((BLK_R, 1), lambda i, j, be: (i, 0)),
        ],
        out_specs=pl.BlockSpec((BLK_R, CP), lambda i, j, be: (i, 0)),
        scratch_shapes=[pltpu.VMEM((BLK_R, CP), jnp.float32)],
    )
    return pl.pallas_call(
        _mlp_body,
        grid_spec=grid_spec,
        out_shape=jax.ShapeDtypeStruct((S, CP), jnp.float32),
    )(bexp_vec, xs, W1, b1.reshape(E, 1, H), W2, b2.reshape(E, 1, C), wcol)


TCH = 16


def _combine_sc_body(outb_hbm, d0_hbm, d1_hbm, y_hbm,
                     d0_v, d1_v, r0a, r1a, r0b, r1b,
                     s0a, s1a, s0b, s1b, swa, swb):
    wid = lax.axis_index("s") * NC + lax.axis_index("c")
    t0 = wid * TW
    pltpu.sync_copy(d0_hbm.at[pl.ds(t0, TW)], d0_v)
    pltpu.sync_copy(d1_hbm.at[pl.ds(t0, TW)], d1_v)

    NCH2 = TW // TCH
    r0s = (r0a, r0b)
    r1s = (r1a, r1b)
    g0sem = (s0a, s0b)
    g1sem = (s1a, s1b)
    wsem = (swa, swb)
    g0 = [None, None]
    g1 = [None, None]
    wr = [None, None]

    def start(ch):
        b = ch % 2
        i0 = d0_v.at[pl.ds(ch * TCH, TCH)]
        i1 = d1_v.at[pl.ds(ch * TCH, TCH)]
        g0[b] = pltpu.async_copy(outb_hbm.at[i0], r0s[b], g0sem[b])
        g1[b] = pltpu.async_copy(outb_hbm.at[i1], r1s[b], g1sem[b])

    def finish(ch):
        b = ch % 2
        g0[b].wait()
        g1[b].wait()
        r0 = r0s[b]
        r1 = r1s[b]

        @plsc.parallel_loop(0, TCH * (CP // 16), unroll=4)
        def _add(t):
            row = lax.shift_right_logical(t, 6)
            col = lax.mul(lax.rem(t, CP // 16), 16)
            r0[row, pl.ds(col, 16)] += r1[row, pl.ds(col, 16)]

        wr[b] = pltpu.async_copy(r0, y_hbm.at[pl.ds(t0 + ch * TCH, TCH)],
                                 wsem[b])

    start(0)
    for ch in range(NCH2):
        b = ch % 2
        if ch + 1 < NCH2:
            if wr[1 - b] is not None:
                wr[1 - b].wait()
            start(ch + 1)
        finish(ch)
    wr[0].wait()
    wr[1].wait()


def _combine_sc(outbuf, d0, d1):
    mesh = plsc.VectorSubcoreMesh(core_axis_name="c", subcore_axis_name="s")
    fn = pl.kernel(
        _combine_sc_body,
        out_type=jax.ShapeDtypeStruct((B, CP), jnp.float32),
        mesh=mesh,
        scratch_types=[
            pltpu.VMEM((TW,), jnp.int32),
            pltpu.VMEM((TW,), jnp.int32),
            pltpu.VMEM((TCH, CP), jnp.float32),
            pltpu.VMEM((TCH, CP), jnp.float32),
            pltpu.VMEM((TCH, CP), jnp.float32),
            pltpu.VMEM((TCH, CP), jnp.float32),
            pltpu.SemaphoreType.DMA,
            pltpu.SemaphoreType.DMA,
            pltpu.SemaphoreType.DMA,
            pltpu.SemaphoreType.DMA,
            pltpu.SemaphoreType.DMA,
            pltpu.SemaphoreType.DMA,
        ],
        compiler_params=pltpu.CompilerParams(needs_layout_passes=False),
    )
    return fn(outbuf, d0, d1)


def kernel(x, gate_W, gate_b, W1, b1, W2, b2):
    idx, cw, aux = _gate_call(x, gate_W, gate_b)
    pairs2d = idx.reshape(RQ, RQ)
    dest2d, bexp2d = _route_call(pairs2d)
    dest_flat = dest2d.reshape(N_PAIR)
    cw_flat = cw.reshape(N_PAIR)
    xs, wbuf = _route_gather_sc(dest_flat, cw_flat, x)
    bexp_vec = bexp2d[0, :NB]
    out_buf = _mlp_call(bexp_vec, xs, W1, b1, W2, b2, wbuf.reshape(S, 1))
    dmat = dest_flat.reshape(B, K)
    y_pad = _combine_sc(out_buf, dmat[:, 0], dmat[:, 1])
    return (y_pad[:, :C], aux[0, 0])
